# baseline (speedup 1.0000x reference)
import jax
import jax.numpy as jnp
from jax.experimental import pallas as pl
from jax.experimental.pallas import tpu as pltpu

N = 1024
B = 2
HID = 1024
LAT = 512
NHEADS = 4
NHID = 64
ALPHA = 0.2
_NEG_LOG2E = -1.4426950408889634
TR = 8
NT = N // TR


def _elu(v):
    return jnp.where(v > 0, v, jnp.exp(v) - 1.0)


def _deepim_body(x_ref, adj_ref, w1_ref, b1_ref, w2_ref, b2_ref, w3_ref, b3_ref,
                 d1_ref, db1_ref, d2_ref, db2_ref, d3_ref, db3_ref, d4_ref, db4_ref,
                 gw_ref, ga_ref, ow_ref, oa_ref, xhat_ref, yhat_ref,
                 nd1_ref, nd2_ref, xcols_ref, hcols_ref):
    x = x_ref[...]

    def _mm(a, w_ref):
        return jnp.dot(a, w_ref[...], preferred_element_type=jnp.float32)

    h = jax.nn.relu(_mm(x, w1_ref) + b1_ref[...])
    h = jax.nn.relu(_mm(h, w2_ref) + b2_ref[...])
    h = jax.nn.relu(_mm(h, w2_ref) + b2_ref[...])
    z = _mm(h, w3_ref) + b3_ref[...]
    h = jax.nn.relu(_mm(z, d1_ref) + db1_ref[...])
    h = jax.nn.relu(_mm(h, d2_ref) + db2_ref[...])
    h = jax.nn.relu(_mm(h, d3_ref) + db3_ref[...])
    x_hat = jax.nn.sigmoid(_mm(h, d4_ref) + db4_ref[...])
    xhat_ref[...] = x_hat

    gw = gw_ref[...]
    ga = ga_ref[...]
    ow = ow_ref[...]
    oa = oa_ref[...]

    xcols_ref[...] = jnp.transpose(x_hat)

    k1 = []
    k2rows = []
    xrows = []
    for b in range(B):
        xr = x_hat[b:b + 1, :]
        for i in range(NHEADS):
            gw_i = gw[i:i + 1, :]
            c1 = jnp.sum(gw_i * ga[i:i + 1, :NHID], axis=1, keepdims=True)
            c2 = jnp.sum(gw_i * ga[i:i + 1, NHID:], axis=1, keepdims=True)
            k1.append(c1 * _NEG_LOG2E)
            k2rows.append(xr * (c2 * _NEG_LOG2E))
            xrows.append(xr)

    def _l1_tile(t, carry):
        r0 = t * TR
        adj_t = adj_ref[pl.ds(r0, TR), :]
        xc_t = xcols_ref[pl.ds(r0, TR), :]
        for m in range(B * NHEADS):
            b = m // NHEADS
            col = xc_t[:, b:b + 1]
            sp = col * k1[m] + k2rows[m]
            e = jnp.exp2(jnp.minimum(sp, ALPHA * sp)) * adj_t
            den = jnp.sum(e, axis=1, keepdims=True)
            num = jnp.sum(e * xrows[m], axis=1, keepdims=True)
            nd1_ref[pl.ds(r0, TR), pl.ds(2 * m, 1)] = num
            nd1_ref[pl.ds(r0, TR), pl.ds(2 * m + 1, 1)] = den
        return carry

    jax.lax.fori_loop(0, NT, _l1_tile, 0, unroll=False)

    nd1 = nd1_ref[...]

    h2s = []
    for b in range(B):
        blocks = []
        for i in range(NHEADS):
            m = b * NHEADS + i
            p = nd1[:, 2 * m:2 * m + 1] / nd1[:, 2 * m + 1:2 * m + 2]
            blocks.append(_elu(_elu(p * gw[i:i + 1, :])))
        xh = jnp.concatenate(blocks, axis=1)
        h2s.append(jnp.dot(xh, ow, preferred_element_type=jnp.float32))

    hcols_ref[...] = jnp.concatenate(h2s, axis=1)
    h2_rows = [jnp.transpose(h2s[b]) for b in range(B)]
    ka = oa[0:1, 0:1] * _NEG_LOG2E
    kb_rows = [h2_rows[b] * (oa[0:1, 1:2] * _NEG_LOG2E) for b in range(B)]

    def _l2_tile(t, carry):
        r0 = t * TR
        adj_t = adj_ref[pl.ds(r0, TR), :]
        hc_t = hcols_ref[pl.ds(r0, TR), :]
        for b in range(B):
            col = hc_t[:, b:b + 1]
            sp = col * ka + kb_rows[b]
            e = jnp.exp2(jnp.minimum(sp, ALPHA * sp)) * adj_t
            den = jnp.sum(e, axis=1, keepdims=True)
            num = jnp.sum(e * h2_rows[b], axis=1, keepdims=True)
            nd2_ref[pl.ds(r0, TR), pl.ds(2 * b, 1)] = num
            nd2_ref[pl.ds(r0, TR), pl.ds(2 * b + 1, 1)] = den
        return carry

    jax.lax.fori_loop(0, NT, _l2_tile, 0, unroll=False)

    nd2 = nd2_ref[...]
    for b in range(B):
        y = _elu(nd2[:, 2 * b:2 * b + 1] / nd2[:, 2 * b + 1:2 * b + 2])
        yhat_ref[b:b + 1, :] = jnp.transpose(y)


def kernel(x, adj, enc_w1, enc_b1, enc_w2, enc_b2, enc_w3, enc_b3,
           dec_w1, dec_b1, dec_w2, dec_b2, dec_w3, dec_b3, dec_w4, dec_b4,
           gat_W, gat_a, out_W, out_a):
    args = (
        x, adj,
        enc_w1, enc_b1.reshape(1, HID), enc_w2, enc_b2.reshape(1, HID),
        enc_w3, enc_b3.reshape(1, LAT),
        dec_w1, dec_b1.reshape(1, LAT), dec_w2, dec_b2.reshape(1, HID),
        dec_w3, dec_b3.reshape(1, HID), dec_w4, dec_b4.reshape(1, N),
        gat_W.reshape(NHEADS, NHID), gat_a.reshape(NHEADS, 2 * NHID),
        out_W, out_a,
    )
    x_hat, y_hat = pl.pallas_call(
        _deepim_body,
        out_shape=(
            jax.ShapeDtypeStruct((B, N), jnp.float32),
            jax.ShapeDtypeStruct((B, N), jnp.float32),
        ),
        scratch_shapes=[
            pltpu.VMEM((N, 2 * B * NHEADS), jnp.float32),
            pltpu.VMEM((N, 2 * B), jnp.float32),
            pltpu.VMEM((N, B), jnp.float32),
            pltpu.VMEM((N, B), jnp.float32),
        ],
    )(*args)
    return x_hat, y_hat


# speedup vs baseline: 2.9593x; 2.9593x over previous
import jax
import jax.numpy as jnp
from jax.experimental import pallas as pl

N = 1024
B = 2
HID = 1024
LAT = 512
NHEADS = 4
NHID = 64
ALPHA = 0.2
_NEG_LOG2E = -1.4426950408889634


def _elu(v):
    return jnp.where(v > 0, v, jnp.exp(v) - 1.0)


def _deepim_body(x_ref, adj_ref, w1_ref, b1_ref, w2_ref, b2_ref, w3_ref, b3_ref,
                 d1_ref, db1_ref, d2_ref, db2_ref, d3_ref, db3_ref, d4_ref, db4_ref,
                 gw_ref, ga_ref, ow_ref, oa_ref, xhat_ref, yhat_ref):
    x = x_ref[...]
    adj = adj_ref[...]

    def _mm(a, w_ref):
        return jnp.dot(a, w_ref[...], preferred_element_type=jnp.float32)

    h = jax.nn.relu(_mm(x, w1_ref) + b1_ref[...])
    h = jax.nn.relu(_mm(h, w2_ref) + b2_ref[...])
    h = jax.nn.relu(_mm(h, w2_ref) + b2_ref[...])
    z = _mm(h, w3_ref) + b3_ref[...]
    h = jax.nn.relu(_mm(z, d1_ref) + db1_ref[...])
    h = jax.nn.relu(_mm(h, d2_ref) + db2_ref[...])
    h = jax.nn.relu(_mm(h, d3_ref) + db3_ref[...])
    x_hat = jax.nn.sigmoid(_mm(h, d4_ref) + db4_ref[...])
    xhat_ref[...] = x_hat

    gw = gw_ref[...]
    ga = ga_ref[...]
    ow = ow_ref[...]
    oa = oa_ref[...]

    for b in range(B):
        xn_row = x_hat[b:b + 1, :]
        xn_col = jnp.transpose(xn_row)

        blocks = []
        for i in range(NHEADS):
            gw_i = gw[i:i + 1, :]
            c1 = jnp.sum(gw_i * ga[i:i + 1, :NHID], axis=1, keepdims=True)
            c2 = jnp.sum(gw_i * ga[i:i + 1, NHID:], axis=1, keepdims=True)
            m1 = xn_col * (c1 * _NEG_LOG2E)
            n1 = xn_row * (c2 * _NEG_LOG2E)
            sp = m1 + n1
            e = jnp.exp2(jnp.minimum(sp, ALPHA * sp)) * adj
            den = jnp.sum(e, axis=1, keepdims=True)
            num = jnp.sum(e * xn_row, axis=1, keepdims=True)
            p = num / den
            blocks.append(_elu(_elu(p * gw_i)))
        xh = jnp.concatenate(blocks, axis=1)

        h2 = jnp.dot(xh, ow, preferred_element_type=jnp.float32)
        h2_row = jnp.transpose(h2)
        m1 = h2 * (oa[0:1, 0:1] * _NEG_LOG2E)
        n1 = h2_row * (oa[0:1, 1:2] * _NEG_LOG2E)
        sp = m1 + n1
        e2 = jnp.exp2(jnp.minimum(sp, ALPHA * sp)) * adj
        den2 = jnp.sum(e2, axis=1, keepdims=True)
        num2 = jnp.sum(e2 * h2_row, axis=1, keepdims=True)
        y = _elu(num2 / den2)
        yhat_ref[b:b + 1, :] = jnp.transpose(y)


def kernel(x, adj, enc_w1, enc_b1, enc_w2, enc_b2, enc_w3, enc_b3,
           dec_w1, dec_b1, dec_w2, dec_b2, dec_w3, dec_b3, dec_w4, dec_b4,
           gat_W, gat_a, out_W, out_a):
    args = (
        x, adj,
        enc_w1, enc_b1.reshape(1, HID), enc_w2, enc_b2.reshape(1, HID),
        enc_w3, enc_b3.reshape(1, LAT),
        dec_w1, dec_b1.reshape(1, LAT), dec_w2, dec_b2.reshape(1, HID),
        dec_w3, dec_b3.reshape(1, HID), dec_w4, dec_b4.reshape(1, N),
        gat_W.reshape(NHEADS, NHID), gat_a.reshape(NHEADS, 2 * NHID),
        out_W, out_a,
    )
    x_hat, y_hat = pl.pallas_call(
        _deepim_body,
        out_shape=(
            jax.ShapeDtypeStruct((B, N), jnp.float32),
            jax.ShapeDtypeStruct((B, N), jnp.float32),
        ),
    )(*args)
    return x_hat, y_hat
